# Initial kernel scaffold; baseline (speedup 1.0000x reference)
import jax
import jax.numpy as jnp
from jax.experimental import pallas as pl


def kernel(C, F, A, SP1, SP0, params, edge_index):
    raise NotImplementedError("write your pallas kernel here")


# speedup vs baseline: 2.1242x; 2.1242x over previous
import functools

import jax
import jax.numpy as jnp
from jax import lax
from jax.experimental import pallas as pl
from jax.experimental.pallas import tpu as pltpu
from jax.experimental.pallas import tpu_sc as plsc

_N_NODES = 100000
_N_EDGES = 1600000
_BN = 5000
_BE = 4000
_CH = 128
_NW = 32
_NPAD = 100096


def _wspec(shape):
    return pl.BlockSpec(shape, lambda i: (0,) * len(shape))


def _rspec(b, w):
    return pl.BlockSpec((b, w), lambda i: (i, 0))


def _enc_node_body(c, f, w1, b1, w2, b2, w3, b3, out):
    h = jnp.maximum(c[...] * w1[0:1, :] + f[...] * w1[1:2, :] + b1[...], 0.0)
    h = jnp.maximum(jnp.dot(h, w2[...], preferred_element_type=jnp.float32) + b2[...], 0.0)
    out[...] = jax.nn.sigmoid(jnp.dot(h, w3[...], preferred_element_type=jnp.float32) + b3[...])


def _enc_edge_body(a, sp1, sp0, w1, b1, w2, b2, w3, b3, out):
    h = jnp.maximum(a[...] * w1[0:1, :] + sp1[...] * w1[1:2, :] + sp0[...] * w1[2:3, :] + b1[...], 0.0)
    h = jnp.maximum(jnp.dot(h, w2[...], preferred_element_type=jnp.float32) + b2[...], 0.0)
    out[...] = jax.nn.sigmoid(jnp.dot(h, w3[...], preferred_element_type=jnp.float32) + b3[...])


def _proj_body(n, wa, wb, oa, ob):
    x = n[...]
    oa[...] = jnp.dot(x, wa[...], preferred_element_type=jnp.float32)
    ob[...] = jnp.dot(x, wb[...], preferred_element_type=jnp.float32)


def _edge_mlp_body(e, gs, gd, w1e, b1, w2, b2, w3, b3, v1h, enew, eproj):
    h = jnp.dot(e[...], w1e[...], preferred_element_type=jnp.float32)
    h = jnp.maximum(h + gs[...] + gd[...] + b1[...], 0.0)
    h = jnp.maximum(jnp.dot(h, w2[...], preferred_element_type=jnp.float32) + b2[...], 0.0)
    en = jax.nn.sigmoid(jnp.dot(h, w3[...], preferred_element_type=jnp.float32) + b3[...])
    enew[...] = en
    eproj[...] = jnp.dot(en, v1h[...], preferred_element_type=jnp.float32)


def _node_mlp_body(n, s0, s1, d0, d1, v1n, c1, v2, c2, v3, c3, out):
    agg = (s0[...] + s1[...]) / jnp.maximum(d0[...] + d1[...], 1.0)
    h = jnp.dot(n[...], v1n[...], preferred_element_type=jnp.float32)
    h = jnp.maximum(h + agg + c1[...], 0.0)
    h = jnp.maximum(jnp.dot(h, v2[...], preferred_element_type=jnp.float32) + c2[...], 0.0)
    out[...] = jax.nn.sigmoid(jnp.dot(h, v3[...], preferred_element_type=jnp.float32) + c3[...])


def _dec_body(e, gs, gd, u1e, u1, w2, u2, w3, u3, out):
    h = jnp.dot(e[...], u1e[...], preferred_element_type=jnp.float32)
    h = jnp.maximum(h + gs[...] + gd[...] + u1[...], 0.0)
    h = jnp.maximum(jnp.dot(h, w2[...], preferred_element_type=jnp.float32) + u2[...], 0.0)
    out[...] = jax.nn.sigmoid(jnp.dot(h, w3[...], preferred_element_type=jnp.float32) + u3[...])


def _enc_nodes(C, F, w1, b1, w2, b2, w3, b3):
    return pl.pallas_call(
        _enc_node_body,
        grid=(_N_NODES // _BN,),
        in_specs=[_rspec(_BN, 1), _rspec(_BN, 1),
                  _wspec((2, 16)), _wspec((1, 16)), _wspec((16, 16)), _wspec((1, 16)),
                  _wspec((16, 64)), _wspec((1, 64))],
        out_specs=_rspec(_BN, 64),
        out_shape=jax.ShapeDtypeStruct((_N_NODES, 64), jnp.float32),
    )(C, F, w1, b1, w2, b2, w3, b3)


def _enc_edges(A, SP1, SP0, w1, b1, w2, b2, w3, b3):
    return pl.pallas_call(
        _enc_edge_body,
        grid=(_N_EDGES // _BE,),
        in_specs=[_rspec(_BE, 1), _rspec(_BE, 1), _rspec(_BE, 1),
                  _wspec((3, 16)), _wspec((1, 16)), _wspec((16, 16)), _wspec((1, 16)),
                  _wspec((16, 64)), _wspec((1, 64))],
        out_specs=_rspec(_BE, 64),
        out_shape=jax.ShapeDtypeStruct((_N_EDGES, 64), jnp.float32),
    )(A, SP1, SP0, w1, b1, w2, b2, w3, b3)


def _proj(n, wa, wb):
    k = wa.shape[1]
    return pl.pallas_call(
        _proj_body,
        grid=(_N_NODES // _BN,),
        in_specs=[_rspec(_BN, 64), _wspec((64, k)), _wspec((64, k))],
        out_specs=[_rspec(_BN, k), _rspec(_BN, k)],
        out_shape=[jax.ShapeDtypeStruct((_N_NODES, k), jnp.float32)] * 2,
    )(n, wa, wb)


def _edge_mlp(e, gs, gd, w1e, b1, w2, b2, w3, b3, v1h):
    return pl.pallas_call(
        _edge_mlp_body,
        grid=(_N_EDGES // _BE,),
        in_specs=[_rspec(_BE, 64), _rspec(_BE, 16), _rspec(_BE, 16),
                  _wspec((64, 16)), _wspec((1, 16)), _wspec((16, 16)), _wspec((1, 16)),
                  _wspec((16, 64)), _wspec((1, 64)), _wspec((64, 16))],
        out_specs=[_rspec(_BE, 64), _rspec(_BE, 16)],
        out_shape=[jax.ShapeDtypeStruct((_N_EDGES, 64), jnp.float32),
                   jax.ShapeDtypeStruct((_N_EDGES, 16), jnp.float32)],
    )(e, gs, gd, w1e, b1, w2, b2, w3, b3, v1h)


def _node_mlp(n, s0, s1, d0, d1, v1n, c1, v2, c2, v3, c3):
    return pl.pallas_call(
        _node_mlp_body,
        grid=(_N_NODES // _BN,),
        in_specs=[_rspec(_BN, 64)] + [_rspec(_BN, 16)] * 4 +
                 [_wspec((64, 16)), _wspec((1, 16)), _wspec((16, 16)), _wspec((1, 16)),
                  _wspec((16, 64)), _wspec((1, 64))],
        out_specs=_rspec(_BN, 64),
        out_shape=jax.ShapeDtypeStruct((_N_NODES, 64), jnp.float32),
    )(n, s0, s1, d0, d1, v1n, c1, v2, c2, v3, c3)


def _decode(e, gs, gd, u1e, u1, w2, u2, w3, u3):
    return pl.pallas_call(
        _dec_body,
        grid=(_N_EDGES // _BE,),
        in_specs=[_rspec(_BE, 64), _rspec(_BE, 64), _rspec(_BE, 64),
                  _wspec((64, 64)), _wspec((1, 64)), _wspec((64, 64)), _wspec((1, 64)),
                  _wspec((64, 1)), _wspec((1, 1))],
        out_specs=_rspec(_BE, 1),
        out_shape=jax.ShapeDtypeStruct((_N_EDGES, 1), jnp.float32),
    )(e, gs, gd, u1e, u1, w2, u2, w3, u3)


def _make_gather(d):
    n_chunks = _N_EDGES // _CH
    iters = (n_chunks + _NW - 1) // _NW
    mesh = plsc.VectorSubcoreMesh(core_axis_name="c", subcore_axis_name="s")

    @functools.partial(
        pl.kernel,
        out_type=[jax.ShapeDtypeStruct((_N_EDGES, d), jnp.float32),
                  jax.ShapeDtypeStruct((_N_EDGES, d), jnp.float32)],
        mesh=mesh,
        compiler_params=pltpu.CompilerParams(use_tc_tiling_on_sc=False),
        scratch_types=[pltpu.VMEM((_CH,), jnp.int32), pltpu.VMEM((_CH,), jnp.int32),
                       pltpu.VMEM((_CH, d), jnp.float32), pltpu.VMEM((_CH, d), jnp.float32),
                       pltpu.SemaphoreType.DMA, pltpu.SemaphoreType.DMA],
    )
    def gather_k(ts_hbm, td_hbm, src_hbm, dst_hbm, gs_hbm, gd_hbm,
                 idx_s, idx_d, buf_s, buf_d, sem_s, sem_d):
        wid = lax.axis_index("s") * 2 + lax.axis_index("c")

        def body(i, carry):
            cid = wid + i * _NW

            @pl.when(cid < n_chunks)
            def _():
                base = cid * _CH
                pltpu.sync_copy(src_hbm.at[pl.ds(base, _CH)], idx_s)
                pltpu.sync_copy(dst_hbm.at[pl.ds(base, _CH)], idx_d)
                cs = pltpu.async_copy(ts_hbm.at[idx_s], buf_s, sem_s)
                cd = pltpu.async_copy(td_hbm.at[idx_d], buf_d, sem_d)
                cs.wait()
                cd.wait()
                pltpu.sync_copy(buf_s, gs_hbm.at[pl.ds(base, _CH)])
                pltpu.sync_copy(buf_d, gd_hbm.at[pl.ds(base, _CH)])

            return carry

        lax.fori_loop(0, iters, body, 0)

    return gather_k


def _make_scatter(with_payload):
    n_chunks = _N_EDGES // _CH
    iters = (n_chunks + _NW - 1) // _NW
    n_zchunks = _NPAD // _CH
    z_iters = (n_zchunks + 15) // 16
    mesh = plsc.VectorSubcoreMesh(core_axis_name="c", subcore_axis_name="s")

    def body_fn(*refs):
        if with_payload:
            (ep_hbm, dst_hbm, out_hbm, idx, pbuf, zbuf, acc) = refs
        else:
            (dst_hbm, out_hbm, idx, pbuf, zbuf, acc) = refs
            ep_hbm = None
        cid = lax.axis_index("c")
        sid = lax.axis_index("s")
        wid = sid * 2 + cid

        def fill(r, carry):
            zbuf[r, :] = jnp.zeros((16,), jnp.float32)
            if not with_payload:
                pbuf[r, :] = jnp.ones((16,), jnp.float32)
            return carry

        lax.fori_loop(0, _CH, fill, 0)

        def zero_chunk(i, carry):
            c = sid + i * 16

            @pl.when(c < n_zchunks)
            def _():
                pltpu.sync_copy(zbuf, acc.at[pl.ds(c * _CH, _CH)])

            return carry

        lax.fori_loop(0, z_iters, zero_chunk, 0)
        plsc.subcore_barrier()

        def scat(i, carry):
            ch = wid + i * _NW

            @pl.when(ch < n_chunks)
            def _():
                base = ch * _CH
                pltpu.sync_copy(dst_hbm.at[pl.ds(base, _CH)], idx)
                if with_payload:
                    pltpu.sync_copy(ep_hbm.at[pl.ds(base, _CH)], pbuf)
                pltpu.sync_copy(pbuf, acc.at[idx], add=True)

            return carry

        lax.fori_loop(0, iters, scat, 0)
        plsc.subcore_barrier()

        def wb(i, carry):
            c = sid + i * 16

            @pl.when(c < n_zchunks)
            def _():
                pltpu.sync_copy(acc.at[pl.ds(c * _CH, _CH)],
                                out_hbm.at[cid, pl.ds(c * _CH, _CH)])

            return carry

        lax.fori_loop(0, z_iters, wb, 0)

    return pl.kernel(
        body_fn,
        out_type=jax.ShapeDtypeStruct((2, _NPAD, 16), jnp.float32),
        mesh=mesh,
        compiler_params=pltpu.CompilerParams(use_tc_tiling_on_sc=False),
        scratch_types=[pltpu.VMEM((_CH,), jnp.int32),
                       pltpu.VMEM((_CH, 16), jnp.float32),
                       pltpu.VMEM((_CH, 16), jnp.float32),
                       pltpu.VMEM_SHARED((_NPAD, 16), jnp.float32)],
    )


_gather16 = _make_gather(16)
_gather64 = _make_gather(64)
_scatter16 = _make_scatter(True)
_degree = _make_scatter(False)


def _b2(v):
    return v.reshape(1, -1)


def kernel(C, F, A, SP1, SP0, params, edge_index):
    src = edge_index[0].astype(jnp.int32)
    dst = edge_index[1].astype(jnp.int32)

    (ew1, eb1), (ew2, eb2), (ew3, eb3) = params['enc_n']
    n = _enc_nodes(C, F, ew1, _b2(eb1), ew2, _b2(eb2), ew3, _b2(eb3))
    (fw1, fb1), (fw2, fb2), (fw3, fb3) = params['enc_e']
    e = _enc_edges(A, SP1, SP0, fw1, _b2(fb1), fw2, _b2(fb2), fw3, _b2(fb3))

    degp = _degree(dst)
    d0 = degp[0, :_N_NODES]
    d1 = degp[1, :_N_NODES]

    for blk in ('c1', 'c3'):
        (w1, b1), (w2, b2), (w3, b3) = params[blk + '_e']
        (v1, c1), (v2, c2), (v3, c3) = params[blk + '_n']
        ps, pd = _proj(n, w1[64:128], w1[128:192])
        gs, gd = _gather16(ps, pd, src, dst)
        e, ep = _edge_mlp(e, gs, gd, w1[:64], _b2(b1), w2, _b2(b2), w3, _b2(b3), v1[64:128])
        sp = _scatter16(ep, dst)
        n = _node_mlp(n, sp[0, :_N_NODES], sp[1, :_N_NODES], d0, d1,
                      v1[:64], _b2(c1), v2, _b2(c2), v3, _b2(c3))

    (u1, ub1), (u2, ub2), (u3, ub3) = params['dec']
    qs, qd = _proj(n, u1[64:128], u1[128:192])
    gs, gd = _gather64(qs, qd, src, dst)
    p = _decode(e, gs, gd, u1[:64], _b2(ub1), u2, _b2(ub2), u3, _b2(ub3))
    return p[:, 0]
